# Initial kernel scaffold; baseline (speedup 1.0000x reference)
import jax
import jax.numpy as jnp
from jax.experimental import pallas as pl


def kernel(edge_attr, W0, W1, W2):
    raise NotImplementedError("write your pallas kernel here")


# speedup vs baseline: 4.9051x; 4.9051x over previous
import functools

import jax
import jax.numpy as jnp
from jax import lax
from jax.experimental import pallas as pl
from jax.experimental.pallas import tpu as pltpu
from jax.experimental.pallas import tpu_sc as plsc

NC = 2
NS = 16
NW = NC * NS
L = 16

CHUNK = 128


def _body(d0, d1, d2, e_total, per_w, ea, w0, w1, w2, out, tbl_hbm,
          w0_v, w1_v, w2_v, tbl_v, ea_v, idx_v, rows_a, rows_b, sem_a, sem_b):
    ncomb = d0 * d1 * d2
    cid = lax.axis_index("c")
    sid = lax.axis_index("s")
    wid = sid * NC + cid

    pltpu.sync_copy(w0, w0_v)
    pltpu.sync_copy(w1, w1_v)
    pltpu.sync_copy(w2, w2_v)

    def build(r, carry):
        a = r // (d1 * d2)
        b = (r // d2) % d1
        c = r % d2
        for ch in range(128 // L):
            sl = pl.ds(ch * L, L)
            tbl_v[r, sl] = w0_v[a, sl] + w1_v[b, sl] + w2_v[c, sl]
        return carry

    lax.fori_loop(0, ncomb, build, 0)
    pltpu.sync_copy(tbl_v, tbl_hbm.at[pl.ds(wid * ncomb, ncomb)])

    base = wid * per_w
    pltpu.sync_copy(ea.at[pl.ds(base * 3, per_w * 3)], ea_v)

    iota3 = lax.iota(jnp.int32, L) * 3
    woff = wid * ncomb

    def fuse(g, carry):
        off = g * (3 * L)
        e0 = plsc.load_gather(ea_v, [iota3 + off])
        e1 = plsc.load_gather(ea_v, [iota3 + (off + 1)])
        e2 = plsc.load_gather(ea_v, [iota3 + (off + 2)])
        idx_v[pl.ds(pl.multiple_of(g * L, L), L)] = \
            (e0 * (d1 * d2) + e1 * d2 + e2) + woff
        return carry

    lax.fori_loop(0, per_w // L, fuse, 0)

    n_full, tail = divmod(per_w, CHUNK)
    n_chunks = n_full + (1 if tail else 0)
    pad = jnp.zeros((L,), jnp.int32) + woff
    for g in range(per_w // L, (n_chunks * CHUNK) // L):
        idx_v[pl.ds(g * L, L)] = pad

    bufs = (rows_a, rows_b)
    sems = (sem_a, sem_b)

    def start(k):
        b = k & 1
        src = tbl_hbm.at[idx_v.at[pl.ds(k * CHUNK, CHUNK)]]
        return pltpu.async_copy(src, bufs[b], sems[b])

    cps = [None, None]
    cps[0] = start(0)
    for k in range(n_chunks):
        b = k & 1
        cps[b].wait()
        if k + 1 < n_chunks:
            cps[1 - b] = start(k + 1)
        rows = CHUNK if (k < n_full) else tail
        pltpu.sync_copy(bufs[b].at[pl.ds(0, rows)],
                        out.at[pl.ds(base + k * CHUNK, rows)])


def _make_kernel(e_total, d0, d1, d2):
    assert e_total % (NW * L) == 0
    per_w = e_total // NW
    ncomb = d0 * d1 * d2
    n_chunks = -(-per_w // CHUNK)
    mesh = plsc.VectorSubcoreMesh(core_axis_name="c", subcore_axis_name="s",
                                  num_cores=NC, num_subcores=NS)
    return pl.kernel(
        functools.partial(_body, d0, d1, d2, e_total, per_w),
        out_type=(jax.ShapeDtypeStruct((e_total, 128), jnp.float32),
                  jax.ShapeDtypeStruct((NW * ncomb, 128), jnp.float32)),
        mesh=mesh,
        compiler_params=pltpu.CompilerParams(needs_layout_passes=False),
        scratch_types=[
            pltpu.VMEM((d0, 128), jnp.float32),
            pltpu.VMEM((d1, 128), jnp.float32),
            pltpu.VMEM((d2, 128), jnp.float32),
            pltpu.VMEM((ncomb, 128), jnp.float32),
            pltpu.VMEM((per_w * 3,), jnp.int32),
            pltpu.VMEM((n_chunks * CHUNK,), jnp.int32),
            pltpu.VMEM((CHUNK, 128), jnp.float32),
            pltpu.VMEM((CHUNK, 128), jnp.float32),
            pltpu.SemaphoreType.DMA,
            pltpu.SemaphoreType.DMA,
        ],
    )


def kernel(edge_attr, W0, W1, W2):
    e_total = edge_attr.shape[0]
    ea_flat = edge_attr.astype(jnp.int32).reshape(-1)
    k = _make_kernel(e_total, W0.shape[0], W1.shape[0], W2.shape[0])
    out, _ = k(ea_flat, W0, W1, W2)
    return out


# speedup vs baseline: 7.0278x; 1.4328x over previous
import functools

import jax
import jax.numpy as jnp
from jax import lax
from jax.experimental import pallas as pl
from jax.experimental.pallas import tpu as pltpu
from jax.experimental.pallas import tpu_sc as plsc

NC = 2
NS = 16
NW = NC * NS
L = 16

CHUNK = 128


def _body(d0, d1, d2, e_total, per_w, ea, w0, w1, w2, out,
          w0_v, w1_v, w2_v, tbl_v, tbl_sh, ea_v, idx_v, rows_a, rows_b,
          sem_a, sem_b):
    ncomb = d0 * d1 * d2
    cid = lax.axis_index("c")
    sid = lax.axis_index("s")
    wid = sid * NC + cid

    @pl.when(sid == 0)
    def _():
        pltpu.sync_copy(w0, w0_v)
        pltpu.sync_copy(w1, w1_v)
        pltpu.sync_copy(w2, w2_v)

        def build(r, carry):
            a = r // (d1 * d2)
            b = (r // d2) % d1
            c = r % d2
            for ch in range(128 // L):
                sl = pl.ds(ch * L, L)
                tbl_v[r, sl] = w0_v[a, sl] + w1_v[b, sl] + w2_v[c, sl]
            return carry

        lax.fori_loop(0, ncomb, build, 0)
        pltpu.sync_copy(tbl_v, tbl_sh)

    base = wid * per_w
    pltpu.sync_copy(ea.at[pl.ds(base * 3, per_w * 3)], ea_v)

    iota3 = lax.iota(jnp.int32, L) * 3

    def fuse(g, carry):
        off = g * (3 * L)
        e0 = plsc.load_gather(ea_v, [iota3 + off])
        e1 = plsc.load_gather(ea_v, [iota3 + (off + 1)])
        e2 = plsc.load_gather(ea_v, [iota3 + (off + 2)])
        idx_v[pl.ds(pl.multiple_of(g * L, L), L)] = \
            e0 * (d1 * d2) + e1 * d2 + e2
        return carry

    lax.fori_loop(0, per_w // L, fuse, 0)

    n_full, tail = divmod(per_w, CHUNK)
    n_chunks = n_full + (1 if tail else 0)
    pad = jnp.zeros((L,), jnp.int32)
    for g in range(per_w // L, (n_chunks * CHUNK) // L):
        idx_v[pl.ds(g * L, L)] = pad

    plsc.subcore_barrier()

    bufs = (rows_a, rows_b)
    sems = (sem_a, sem_b)

    def start(k):
        b = k & 1
        src = tbl_sh.at[idx_v.at[pl.ds(k * CHUNK, CHUNK)]]
        return pltpu.async_copy(src, bufs[b], sems[b])

    cps = [None, None]
    cps[0] = start(0)
    for k in range(n_chunks):
        b = k & 1
        cps[b].wait()
        if k + 1 < n_chunks:
            cps[1 - b] = start(k + 1)
        rows = CHUNK if (k < n_full) else tail
        pltpu.sync_copy(bufs[b].at[pl.ds(0, rows)],
                        out.at[pl.ds(base + k * CHUNK, rows)])


def _make_kernel(e_total, d0, d1, d2):
    assert e_total % (NW * L) == 0
    per_w = e_total // NW
    ncomb = d0 * d1 * d2
    n_chunks = -(-per_w // CHUNK)
    mesh = plsc.VectorSubcoreMesh(core_axis_name="c", subcore_axis_name="s",
                                  num_cores=NC, num_subcores=NS)
    return pl.kernel(
        functools.partial(_body, d0, d1, d2, e_total, per_w),
        out_type=jax.ShapeDtypeStruct((e_total, 128), jnp.float32),
        mesh=mesh,
        compiler_params=pltpu.CompilerParams(needs_layout_passes=False),
        scratch_types=[
            pltpu.VMEM((d0, 128), jnp.float32),
            pltpu.VMEM((d1, 128), jnp.float32),
            pltpu.VMEM((d2, 128), jnp.float32),
            pltpu.VMEM((ncomb, 128), jnp.float32),
            pltpu.VMEM_SHARED((ncomb, 128), jnp.float32),
            pltpu.VMEM((per_w * 3,), jnp.int32),
            pltpu.VMEM((n_chunks * CHUNK,), jnp.int32),
            pltpu.VMEM((CHUNK, 128), jnp.float32),
            pltpu.VMEM((CHUNK, 128), jnp.float32),
            pltpu.SemaphoreType.DMA,
            pltpu.SemaphoreType.DMA,
        ],
    )


def kernel(edge_attr, W0, W1, W2):
    e_total = edge_attr.shape[0]
    ea_flat = edge_attr.astype(jnp.int32).reshape(-1)
    k = _make_kernel(e_total, W0.shape[0], W1.shape[0], W2.shape[0])
    return k(ea_flat, W0, W1, W2)


# speedup vs baseline: 17.9343x; 2.5519x over previous
import functools

import jax
import jax.numpy as jnp
from jax import lax
from jax.experimental import pallas as pl
from jax.experimental.pallas import tpu as pltpu
from jax.experimental.pallas import tpu_sc as plsc

NC = 2
NS = 16
NW = NC * NS
L = 16

CHUNK = 128


def _body(d0, d1, d2, e_total, per_w, ea0, ea1, ea2, w0, w1, w2, out,
          w0_v, w1_v, w2_v, tbl_v, tbl_sh, e0_v, e1_v, e2_v, idx_v,
          rows_a, rows_b, sem_a, sem_b):
    ncomb = d0 * d1 * d2
    cid = lax.axis_index("c")
    sid = lax.axis_index("s")
    wid = sid * NC + cid

    @pl.when(sid == 0)
    def _():
        pltpu.sync_copy(w0, w0_v)
        pltpu.sync_copy(w1, w1_v)
        pltpu.sync_copy(w2, w2_v)

        def build(r, carry):
            a = r // (d1 * d2)
            b = (r // d2) % d1
            c = r % d2
            for ch in range(128 // L):
                sl = pl.ds(ch * L, L)
                tbl_v[r, sl] = w0_v[a, sl] + w1_v[b, sl] + w2_v[c, sl]
            return carry

        lax.fori_loop(0, ncomb, build, 0)
        pltpu.sync_copy(tbl_v, tbl_sh)

    base = wid * per_w
    pltpu.sync_copy(ea0.at[pl.ds(base, per_w)], e0_v)
    pltpu.sync_copy(ea1.at[pl.ds(base, per_w)], e1_v)
    pltpu.sync_copy(ea2.at[pl.ds(base, per_w)], e2_v)

    def fuse(g, carry):
        sl = pl.ds(pl.multiple_of(g * L, L), L)
        idx_v[sl] = e0_v[sl] * (d1 * d2) + e1_v[sl] * d2 + e2_v[sl]
        return carry

    lax.fori_loop(0, per_w // L, fuse, 0)

    n_full, tail = divmod(per_w, CHUNK)
    n_chunks = n_full + (1 if tail else 0)
    pad = jnp.zeros((L,), jnp.int32)
    for g in range(per_w // L, (n_chunks * CHUNK) // L):
        idx_v[pl.ds(g * L, L)] = pad

    plsc.subcore_barrier()

    bufs = (rows_a, rows_b)
    sems = (sem_a, sem_b)

    def start(k):
        b = k & 1
        src = tbl_sh.at[idx_v.at[pl.ds(k * CHUNK, CHUNK)]]
        return pltpu.async_copy(src, bufs[b], sems[b])

    cps = [None, None]
    cps[0] = start(0)
    for k in range(n_chunks):
        b = k & 1
        cps[b].wait()
        if k + 1 < n_chunks:
            cps[1 - b] = start(k + 1)
        rows = CHUNK if (k < n_full) else tail
        pltpu.sync_copy(bufs[b].at[pl.ds(0, rows)],
                        out.at[pl.ds(base + k * CHUNK, rows)])


def _make_kernel(e_total, d0, d1, d2):
    assert e_total % (NW * L) == 0
    per_w = e_total // NW
    ncomb = d0 * d1 * d2
    n_chunks = -(-per_w // CHUNK)
    mesh = plsc.VectorSubcoreMesh(core_axis_name="c", subcore_axis_name="s",
                                  num_cores=NC, num_subcores=NS)
    return pl.kernel(
        functools.partial(_body, d0, d1, d2, e_total, per_w),
        out_type=jax.ShapeDtypeStruct((e_total, 128), jnp.float32),
        mesh=mesh,
        compiler_params=pltpu.CompilerParams(needs_layout_passes=False),
        scratch_types=[
            pltpu.VMEM((d0, 128), jnp.float32),
            pltpu.VMEM((d1, 128), jnp.float32),
            pltpu.VMEM((d2, 128), jnp.float32),
            pltpu.VMEM((ncomb, 128), jnp.float32),
            pltpu.VMEM_SHARED((ncomb, 128), jnp.float32),
            pltpu.VMEM((per_w,), jnp.int32),
            pltpu.VMEM((per_w,), jnp.int32),
            pltpu.VMEM((per_w,), jnp.int32),
            pltpu.VMEM((n_chunks * CHUNK,), jnp.int32),
            pltpu.VMEM((CHUNK, 128), jnp.float32),
            pltpu.VMEM((CHUNK, 128), jnp.float32),
            pltpu.SemaphoreType.DMA,
            pltpu.SemaphoreType.DMA,
        ],
    )


def kernel(edge_attr, W0, W1, W2):
    e_total = edge_attr.shape[0]
    ea = edge_attr.astype(jnp.int32)
    k = _make_kernel(e_total, W0.shape[0], W1.shape[0], W2.shape[0])
    return k(ea[:, 0], ea[:, 1], ea[:, 2], W0, W1, W2)


# speedup vs baseline: 18.2581x; 1.0181x over previous
import functools

import jax
import jax.numpy as jnp
from jax import lax
from jax.experimental import pallas as pl
from jax.experimental.pallas import tpu as pltpu
from jax.experimental.pallas import tpu_sc as plsc

NC = 2
NS = 16
NW = NC * NS
L = 16

CHUNK = 128


def _body(d0, d1, d2, e_total, per_w, ea0, ea1, ea2, w0, w1, w2, out,
          w0_v, w1_v, w2_v, tbl_v, tbl_sh, e0_v, e1_v, e2_v, idx_v,
          rows_a, rows_b, sem_a, sem_b, sem_c, sem_d, sem_e):
    ncomb = d0 * d1 * d2
    cid = lax.axis_index("c")
    sid = lax.axis_index("s")
    wid = sid * NC + cid
    base = wid * per_w
    n_full, tail = divmod(per_w, CHUNK)
    n_chunks = n_full + (1 if tail else 0)
    gpc = CHUNK // L

    col_cps = [
        pltpu.async_copy(ea0.at[pl.ds(base, per_w)], e0_v.at[pl.ds(0, per_w)], sem_c),
        pltpu.async_copy(ea1.at[pl.ds(base, per_w)], e1_v.at[pl.ds(0, per_w)], sem_c),
        pltpu.async_copy(ea2.at[pl.ds(base, per_w)], e2_v.at[pl.ds(0, per_w)], sem_c),
    ]
    zeros = jnp.zeros((L,), jnp.int32)
    for g in range(per_w // L, (n_chunks * CHUNK) // L):
        sl = pl.ds(g * L, L)
        e0_v[sl] = zeros
        e1_v[sl] = zeros
        e2_v[sl] = zeros

    @pl.when(sid == 0)
    def _():
        pltpu.sync_copy(w0, w0_v)
        pltpu.sync_copy(w1, w1_v)
        pltpu.sync_copy(w2, w2_v)

        def build(r, carry):
            a = r // (d1 * d2)
            b = (r // d2) % d1
            c = r % d2
            for ch in range(128 // L):
                sl = pl.ds(ch * L, L)
                tbl_v[r, sl] = w0_v[a, sl] + w1_v[b, sl] + w2_v[c, sl]
            return carry

        lax.fori_loop(0, ncomb, build, 0)
        pltpu.sync_copy(tbl_v, tbl_sh)

    plsc.subcore_barrier()
    for cp in col_cps:
        cp.wait()

    def fuse(g, carry):
        sl = pl.ds(pl.multiple_of(g * L, L), L)
        idx_v[sl] = e0_v[sl] * (d1 * d2) + e1_v[sl] * d2 + e2_v[sl]
        return carry

    bufs = (rows_a, rows_b)
    sems = (sem_a, sem_b)

    def start(k):
        b = k & 1
        src = tbl_sh.at[idx_v.at[pl.ds(k * CHUNK, CHUNK)]]
        return pltpu.async_copy(src, bufs[b], sems[b])

    wsems = (sem_d, sem_e)

    lax.fori_loop(0, gpc, fuse, 0)
    cps = [None, None]
    wcps = [None, None]
    cps[0] = start(0)
    for k in range(n_chunks):
        b = k & 1
        if k + 1 < n_chunks:
            lax.fori_loop((k + 1) * gpc, (k + 2) * gpc, fuse, 0)
        cps[b].wait()
        if k + 1 < n_chunks:
            if wcps[1 - b] is not None:
                wcps[1 - b].wait()
            cps[1 - b] = start(k + 1)
        rows = CHUNK if (k < n_full) else tail
        wcps[b] = pltpu.async_copy(bufs[b].at[pl.ds(0, rows)],
                                   out.at[pl.ds(base + k * CHUNK, rows)],
                                   wsems[b])
    for w in wcps:
        if w is not None:
            w.wait()


def _make_kernel(e_total, d0, d1, d2):
    assert e_total % (NW * L) == 0
    per_w = e_total // NW
    ncomb = d0 * d1 * d2
    n_chunks = -(-per_w // CHUNK)
    mesh = plsc.VectorSubcoreMesh(core_axis_name="c", subcore_axis_name="s",
                                  num_cores=NC, num_subcores=NS)
    return pl.kernel(
        functools.partial(_body, d0, d1, d2, e_total, per_w),
        out_type=jax.ShapeDtypeStruct((e_total, 128), jnp.float32),
        mesh=mesh,
        compiler_params=pltpu.CompilerParams(needs_layout_passes=False),
        scratch_types=[
            pltpu.VMEM((d0, 128), jnp.float32),
            pltpu.VMEM((d1, 128), jnp.float32),
            pltpu.VMEM((d2, 128), jnp.float32),
            pltpu.VMEM((ncomb, 128), jnp.float32),
            pltpu.VMEM_SHARED((ncomb, 128), jnp.float32),
            pltpu.VMEM((n_chunks * CHUNK,), jnp.int32),
            pltpu.VMEM((n_chunks * CHUNK,), jnp.int32),
            pltpu.VMEM((n_chunks * CHUNK,), jnp.int32),
            pltpu.VMEM((n_chunks * CHUNK,), jnp.int32),
            pltpu.VMEM((CHUNK, 128), jnp.float32),
            pltpu.VMEM((CHUNK, 128), jnp.float32),
            pltpu.SemaphoreType.DMA,
            pltpu.SemaphoreType.DMA,
            pltpu.SemaphoreType.DMA,
            pltpu.SemaphoreType.DMA,
            pltpu.SemaphoreType.DMA,
        ],
    )


def kernel(edge_attr, W0, W1, W2):
    e_total = edge_attr.shape[0]
    ea = edge_attr.astype(jnp.int32)
    k = _make_kernel(e_total, W0.shape[0], W1.shape[0], W2.shape[0])
    return k(ea[:, 0], ea[:, 1], ea[:, 2], W0, W1, W2)
